# Initial kernel scaffold; baseline (speedup 1.0000x reference)
import jax
import jax.numpy as jnp
from jax.experimental import pallas as pl


def kernel(user_embed, item_embed, edge_index, edge_values):
    raise NotImplementedError("write your pallas kernel here")


# speedup vs baseline: 7.0780x; 7.0780x over previous
import functools

import jax
import jax.numpy as jnp
from jax import lax
from jax.experimental import pallas as pl
from jax.experimental.pallas import tpu as pltpu
from jax.experimental.pallas import tpu_sc as plsc

N_USERS = 50000
N_ITEMS = 50000
N = N_USERS + N_ITEMS
D = 32
H = 16
E = 1600000
N_HOPS = 3
EDGE_DROP = 0.5
MESS_DROP = 0.1

NC = 2
NT = 16
ET = E // NT
W = 800
NW = ET // W
K = 100
NK = W // K
NP = 102400
RT = NP // NT
RV = 320
NEV = RT // RV
ROWS_PER_WIN = W // K


def _sc_body(init_ref, srcg_ref, dst_ref, v3_ref, mask3_ref,
             o1_ref, o2_ref, o3_ref,
             acc, gidx, dstb, vb, rows, maskb, evb, sem):
    c = lax.axis_index("c")
    s = lax.axis_index("s")
    coff = c * NP
    tile_rbase = s * RT

    for hop in range(N_HOPS):
        src_tab = (init_ref, o1_ref, o2_ref)[hop]
        out_tab = (o1_ref, o2_ref, o3_ref)[hop]

        def zbody(i, carry):
            evb[i, :] = jnp.zeros((H,), jnp.float32)
            return carry
        lax.fori_loop(0, RV, zbody, 0)
        for k in range(NEV):
            pltpu.sync_copy(evb, acc.at[pl.ds(tile_rbase + k * RV, RV)])
        plsc.subcore_barrier()

        def wbody(w, carry):
            irow = c * (E // K) + s * (ET // K) + w * ROWS_PER_WIN
            pltpu.sync_copy(srcg_ref.at[pl.ds(irow, ROWS_PER_WIN)], gidx)
            drow = s * (ET // K) + w * ROWS_PER_WIN
            pltpu.sync_copy(dst_ref.at[pl.ds(drow, ROWS_PER_WIN)], dstb)
            vbase = hop * E + s * ET + w * W
            pltpu.sync_copy(v3_ref.at[pl.ds(vbase, W)], vb)

            descs = [
                pltpu.async_copy(src_tab.at[gidx.at[j]],
                                 rows.at[pl.ds(j * K, K)], sem)
                for j in range(NK)
            ]
            for dsc in descs:
                dsc.wait()

            def sbody(i, carry2):
                vvec = vb[pl.ds(i * 16, 16)]
                for e in range(16):
                    r = i * 16 + e
                    rows[r, :] = rows[r, :] * vvec[e]
                return carry2
            lax.fori_loop(0, W // 16, sbody, 0)

            for j in range(NK):
                pltpu.sync_copy(rows.at[pl.ds(j * K, K)],
                                acc.at[dstb.at[j]], add=True)
            return carry
        lax.fori_loop(0, NW, wbody, 0)
        plsc.subcore_barrier()

        for k in range(NEV):
            r0 = tile_rbase + k * RV
            pltpu.sync_copy(acc.at[pl.ds(r0, RV)], evb)
            pltpu.sync_copy(
                mask3_ref.at[pl.ds(hop * NC * NP + coff + r0, RV)], maskb)

            def mbody(i, carry):
                evb[i, :] = evb[i, :] * maskb[i, :]
                return carry
            lax.fori_loop(0, RV, mbody, 0)
            pltpu.sync_copy(evb, out_tab.at[pl.ds(coff + r0, RV)])
        plsc.subcore_barrier()


@functools.partial(
    pl.kernel,
    out_type=[jax.ShapeDtypeStruct((NC * NP, H), jnp.float32)] * N_HOPS,
    mesh=plsc.VectorSubcoreMesh(core_axis_name="c", subcore_axis_name="s",
                                num_cores=NC, num_subcores=NT),
    scratch_types=[
        pltpu.VMEM_SHARED((NP, H), jnp.float32),
        pltpu.VMEM((ROWS_PER_WIN, K), jnp.int32),
        pltpu.VMEM((ROWS_PER_WIN, K), jnp.int32),
        pltpu.VMEM((W,), jnp.float32),
        pltpu.VMEM((W, H), jnp.float32),
        pltpu.VMEM((RV, H), jnp.float32),
        pltpu.VMEM((RV, H), jnp.float32),
        pltpu.SemaphoreType.DMA,
    ],
    compiler_params=pltpu.CompilerParams(use_tc_tiling_on_sc=False),
)
def _graph_conv_sc(init_ref, srcg_ref, dst_ref, v3_ref, mask3_ref,
                   o1_ref, o2_ref, o3_ref,
                   acc, gidx, dstb, vb, rows, maskb, evb, sem):
    _sc_body(init_ref, srcg_ref, dst_ref, v3_ref, mask3_ref,
             o1_ref, o2_ref, o3_ref,
             acc, gidx, dstb, vb, rows, maskb, evb, sem)


def kernel(user_embed, item_embed, edge_index, edge_values):
    all_embed = jnp.concatenate([user_embed, item_embed], axis=0)
    zpad = jnp.zeros((NP - N, H), jnp.float32)
    init_tab = jnp.concatenate(
        [all_embed[:, :H], zpad, all_embed[:, H:], zpad], axis=0)

    dst = edge_index[0]
    src = edge_index[1]

    base_key = jax.random.key(42)
    vs, masks = [], []
    for hop in range(N_HOPS):
        ke, km = jax.random.split(jax.random.fold_in(base_key, hop))
        u = jax.random.uniform(ke, (E,), dtype=jnp.float32)
        keep = jnp.floor(EDGE_DROP + u)
        vs.append(edge_values * keep * (1.0 / (1.0 - EDGE_DROP)))
        m = (jax.random.uniform(km, (N, D)) >= MESS_DROP).astype(jnp.float32)
        m = m * (1.0 / (1.0 - MESS_DROP))
        mpad = jnp.zeros((NP - N, H), jnp.float32)
        masks.append(jnp.concatenate([m[:, :H], mpad, m[:, H:], mpad], axis=0))
    v3 = jnp.concatenate(vs)
    mask3 = jnp.concatenate(masks)

    srcg = jnp.concatenate([src, src + NP]).reshape(NC * E // K, K)
    dst2d = dst.reshape(E // K, K)

    o1, o2, o3 = _graph_conv_sc(init_tab, srcg, dst2d, v3, mask3)

    def halves(tab):
        return jnp.concatenate([tab[:N], tab[NP:NP + N]], axis=1)

    embs = jnp.stack(
        [all_embed, halves(o1), halves(o2), halves(o3)], axis=1)
    return embs[:N_USERS], embs[N_USERS:]
